# initial kernel scaffold (unmeasured)
import jax
import jax.numpy as jnp
from jax import lax
from jax.experimental import pallas as pl
from jax.experimental.pallas import tpu as pltpu


def kernel(
    x,
):
    def body(*refs):
        pass

    out_shape = jax.ShapeDtypeStruct(..., jnp.float32)
    return pl.pallas_call(body, out_shape=out_shape)(...)


# baseline (device time: 21321 ns/iter reference)
import jax
import jax.numpy as jnp
from jax import lax
from jax.experimental import pallas as pl
from jax.experimental.pallas import tpu as pltpu

N_DEV = 32
DISTS = (1, 2, 4, 8, 16)
N_ROUNDS = 1 + len(DISTS)


def kernel(x):
    m, n = x.shape

    def body(x_ref, out_ref, acc_ref, v_ref, comm_ref, send_sems, recv_sems):
        my = lax.axis_index("i")

        barrier = pltpu.get_barrier_semaphore()
        for d in DISTS:
            src = lax.rem(my - d + N_DEV, N_DEV)
            pl.semaphore_signal(
                barrier, inc=1,
                device_id=(src,), device_id_type=pl.DeviceIdType.MESH,
            )
        pl.semaphore_wait(barrier, len(DISTS))

        acc_ref[...] = x_ref[...].astype(jnp.float32)
        s = 1
        while s < m:
            prev = acc_ref[pl.ds(0, m - s), :]
            cur = acc_ref[pl.ds(s, m - s), :]
            acc_ref[pl.ds(s, m - s), :] = cur * prev
            s *= 2

        v_ref[...] = acc_ref[pl.ds(m - 1, 1), :]

        rounds = [(0, 1, True)] + [
            (r + 1, d, False) for r, d in enumerate(DISTS)
        ]
        for slot, dist, is_shift in rounds:
            dst = lax.rem(my + dist, N_DEV)
            rdma = pltpu.make_async_remote_copy(
                src_ref=v_ref,
                dst_ref=comm_ref.at[slot],
                send_sem=send_sems.at[slot],
                recv_sem=recv_sems.at[slot],
                device_id=(dst,),
                device_id_type=pl.DeviceIdType.MESH,
            )
            rdma.start()
            rdma.wait()
            recv = comm_ref[slot, :, :]
            if is_shift:
                v_ref[...] = jnp.where(my >= 1, recv, jnp.ones_like(recv))
            else:
                v_ref[...] = jnp.where(my >= dist, v_ref[...] * recv, v_ref[...])

        out_ref[...] = acc_ref[...] * v_ref[...]

    return pl.pallas_call(
        body,
        out_shape=jax.ShapeDtypeStruct((m, n), jnp.float32),
        in_specs=[pl.BlockSpec(memory_space=pltpu.VMEM)],
        out_specs=pl.BlockSpec(memory_space=pltpu.VMEM),
        scratch_shapes=[
            pltpu.VMEM((m, n), jnp.float32),
            pltpu.VMEM((1, n), jnp.float32),
            pltpu.VMEM((N_ROUNDS, 1, n), jnp.float32),
            pltpu.SemaphoreType.DMA((N_ROUNDS,)),
            pltpu.SemaphoreType.DMA((N_ROUNDS,)),
        ],
        compiler_params=pltpu.CompilerParams(collective_id=0),
    )(x)


# device time: 14407 ns/iter; 1.4799x vs baseline; 1.4799x over previous
import jax
import jax.numpy as jnp
from jax import lax
from jax.experimental import pallas as pl
from jax.experimental.pallas import tpu as pltpu

N_DEV = 32


def kernel(x):
    m, n = x.shape

    def body(x_ref, out_ref, acc_ref, v_ref, comm_ref, send_sems, recv_sems):
        my = lax.axis_index("i")

        barrier = pltpu.get_barrier_semaphore()
        for j in range(1, N_DEV):
            src = lax.rem(my - j + N_DEV, N_DEV)
            pl.semaphore_signal(
                barrier, inc=1,
                device_id=(src,), device_id_type=pl.DeviceIdType.MESH,
            )
        pl.semaphore_wait(barrier, N_DEV - 1)

        xf = x_ref[...].astype(jnp.float32)
        acc_ref[...] = xf
        t = xf
        size = m
        while size > 1:
            half = size // 2
            t = t[:half] * t[half:size]
            size = half
        v_ref[...] = t

        rdmas = []
        for j in range(1, N_DEV):
            dst = lax.rem(my + j, N_DEV)
            rdma = pltpu.make_async_remote_copy(
                src_ref=v_ref,
                dst_ref=comm_ref.at[j],
                send_sem=send_sems.at[j],
                recv_sem=recv_sems.at[j],
                device_id=(dst,),
                device_id_type=pl.DeviceIdType.MESH,
            )
            rdma.start()
            rdmas.append(rdma)

        s = 1
        while s < m:
            prev = acc_ref[pl.ds(0, m - s), :]
            cur = acc_ref[pl.ds(s, m - s), :]
            acc_ref[pl.ds(s, m - s), :] = cur * prev
            s *= 2

        for rdma in rdmas:
            rdma.wait()

        vals = comm_ref[:, 0, :]
        row = lax.broadcasted_iota(jnp.int32, (N_DEV, n), 0)
        srcidx = lax.rem(my - row + N_DEV, N_DEV)
        masked = jnp.where(srcidx < my, vals, jnp.ones_like(vals))
        size = N_DEV
        while size > 1:
            half = size // 2
            masked = masked[:half] * masked[half:size]
            size = half
        prefix = masked

        out_ref[...] = acc_ref[...] * prefix

    return pl.pallas_call(
        body,
        out_shape=jax.ShapeDtypeStruct((m, n), jnp.float32),
        in_specs=[pl.BlockSpec(memory_space=pltpu.VMEM)],
        out_specs=pl.BlockSpec(memory_space=pltpu.VMEM),
        scratch_shapes=[
            pltpu.VMEM((m, n), jnp.float32),
            pltpu.VMEM((1, n), jnp.float32),
            pltpu.VMEM((N_DEV, 1, n), jnp.float32),
            pltpu.SemaphoreType.DMA((N_DEV,)),
            pltpu.SemaphoreType.DMA((N_DEV,)),
        ],
        compiler_params=pltpu.CompilerParams(collective_id=0),
    )(x)
